# baseline (speedup 1.0000x reference)
import functools

import jax
import jax.numpy as jnp
from jax import lax
from jax.experimental import pallas as pl
from jax.experimental.pallas import tpu as pltpu
from jax.experimental.pallas import tpu_sc as plsc

_B = 16384
_D = 64
_NT = 5
_G = 16
_Q = 4
_H = 256


@functools.cache
def _build():
    info = plsc.get_sparse_core_info()
    nc, ns = info.num_cores, info.num_subcores
    nw = nc * ns
    b_per_w = _B // nw
    n_h = b_per_w // _H
    hgroups = _H // _G
    mesh = plsc.VectorSubcoreMesh(core_axis_name="c", subcore_axis_name="s")

    @functools.partial(
        pl.kernel,
        mesh=mesh,
        out_type=jax.ShapeDtypeStruct((_NT, nw, n_h, _H, _D), jnp.float32),
        compiler_params=pltpu.CompilerParams(use_tc_tiling_on_sc=True,
                                             needs_layout_passes=False),
        scratch_types=(
            [pltpu.VMEM((_H,), jnp.int32)]
            + [pltpu.SMEM((_H,), jnp.int32)]
            + [pltpu.VMEM((_H, _D), jnp.float32)]
            + [pltpu.VMEM((_G, 8, _D), jnp.float32) for _ in range(_Q)]
            + [pltpu.SemaphoreType.DMA for _ in range(_Q)]
            + [pltpu.SemaphoreType.DMA]
            + [pltpu.SemaphoreType.DMA]
        ),
    )
    def node_embedding(idx_h, w_cat, w_sub, w_elem, w_brand, w_item, out_h,
                       idx_v, idx_s, stage, *rest):
        blks = rest[:_Q]
        gsems = rest[_Q:2 * _Q]
        isem = rest[2 * _Q]
        osem = rest[2 * _Q + 1]
        tabs = [w_cat, w_sub, w_elem, w_brand, w_item]
        wid = lax.axis_index("s") * nc + lax.axis_index("c")

        def gather_group(tab, q, g):
            vec = idx_v[pl.ds(g * _G, _G)]
            for u in range(_G):
                v = vec[u]
                idx_s[g * _G + u] = v & 7
                pltpu.async_copy(tab.at[v >> 3], blks[q].at[u], gsems[q])

        def gdrain(q):
            pltpu.make_async_copy(tabs[0].at[pl.ds(0, _G)], blks[q],
                                  gsems[q]).wait()

        def extract(q, g):
            for u in range(_G):
                r = idx_s[g * _G + u]
                for k in range(_D // 16):
                    stage[g * _G + u, pl.ds(16 * k, 16)] = (
                        blks[q][u, r, pl.ds(16 * k, 16)])

        for t in range(_NT):
            tab = tabs[t]

            def hbody(h, _, tab=tab, t=t):
                pltpu.async_copy(
                    idx_h.at[wid, pl.ds(t * b_per_w + h * _H, _H)], idx_v,
                    isem).wait()
                for m in range(_Q - 1):
                    gather_group(tab, m, m)

                def quad(j, _, tab=tab):
                    g0 = j * _Q
                    for m in range(_Q):
                        g = g0 + m

                        @pl.when(g + _Q - 1 < hgroups)
                        def _():
                            gather_group(tab, (m + _Q - 1) % _Q,
                                         g + _Q - 1)
                        gdrain(m)
                        extract(m, g)
                    return ()

                lax.fori_loop(0, hgroups // _Q, quad, ())
                pltpu.async_copy(
                    stage, out_h.at[t, wid, h], osem).wait()
                return ()

            lax.fori_loop(0, n_h, hbody, ())

    return node_embedding, nw, b_per_w


def kernel(categories, sub_categories, elements, brands, product_id_remapped,
           W_cat, W_sub, W_elem, W_brand, W_item):
    fn, nw, b_per_w = _build()
    idx = jnp.stack([categories, sub_categories, elements, brands,
                     product_id_remapped]).astype(jnp.int32)
    idx = idx.reshape(_NT, nw, b_per_w).transpose(1, 0, 2)
    idx = idx.reshape(nw, _NT * b_per_w)
    parts = fn(idx,
               W_cat.reshape(-1, 8, _D), W_sub.reshape(-1, 8, _D),
               W_elem.reshape(-1, 8, _D), W_brand.reshape(-1, 8, _D),
               W_item.reshape(-1, 8, _D))
    return parts.reshape(_NT, _B, _D).transpose(1, 0, 2).reshape(
        _B, _NT * _D)
